# baseline (speedup 1.0000x reference)
import jax
import jax.numpy as jnp
from jax.experimental import pallas as pl
from jax.experimental.pallas import tpu as pltpu

_BM = 200


def _gcn_body(x_ref, adj_a_ref, adj_b_ref, w_ref, b_ref, out_ref, support_ref):
    @pl.when(pl.program_id(0) == 0)
    def _():
        support_ref[...] = jnp.dot(
            x_ref[...], w_ref[...], preferred_element_type=jnp.float32
        )

    s = support_ref[...]
    b = b_ref[...]
    out_ref[:_BM, :] = (
        jnp.dot(adj_a_ref[...], s, preferred_element_type=jnp.float32) + b
    )
    out_ref[_BM:, :] = (
        jnp.dot(adj_b_ref[...], s, preferred_element_type=jnp.float32) + b
    )


def kernel(input, adj, weight, bias):
    n, k = input.shape
    m = adj.shape[0]
    f = weight.shape[1]
    bias2 = bias.reshape(1, f)

    return pl.pallas_call(
        _gcn_body,
        grid=(m // (2 * _BM),),
        in_specs=[
            pl.BlockSpec((n, k), lambda i: (0, 0)),
            pl.BlockSpec((_BM, n), lambda i: (2 * i, 0)),
            pl.BlockSpec((_BM, n), lambda i: (2 * i + 1, 0)),
            pl.BlockSpec((k, f), lambda i: (0, 0)),
            pl.BlockSpec((1, f), lambda i: (0, 0)),
        ],
        out_specs=pl.BlockSpec((2 * _BM, f), lambda i: (i, 0)),
        out_shape=jax.ShapeDtypeStruct((m, f), jnp.float32),
        scratch_shapes=[pltpu.VMEM((n, f), jnp.float32)],
    )(input, adj, adj, weight, bias2)
